# baseline (speedup 1.0000x reference)
import functools

import jax
import jax.numpy as jnp
from jax import lax
from jax.experimental import pallas as pl
from jax.experimental.pallas import tpu as pltpu
from jax.experimental.pallas import tpu_sc as plsc

_NC = 2
_NS = 16
_SUB = 16
_LANES = 16


def _make_sc_kernel(b, l, d, dtype):
    nw = _NC * _NS
    l_per_w = l // nw
    nsub = l_per_w // _SUB

    mesh = plsc.VectorSubcoreMesh(core_axis_name="c", subcore_axis_name="s")

    @functools.partial(
        pl.kernel,
        mesh=mesh,
        out_type=jax.ShapeDtypeStruct((b * l, d), dtype),
        scratch_types=[
            pltpu.VMEM((_SUB, d), dtype),
            pltpu.VMEM((_SUB, d), dtype),
        ],
    )
    def k(x_hbm, pe_hbm, out_hbm, pe_buf, x_buf):
        c = lax.axis_index("c")
        s = lax.axis_index("s")
        wid = s * _NC + c
        lbase = wid * l_per_w

        def sub_body(j, _):
            l0 = lbase + j * _SUB
            pltpu.sync_copy(pe_hbm.at[pl.ds(l0, _SUB)], pe_buf)

            def batch_body(bi, _):
                r0 = bi * l + l0
                pltpu.sync_copy(x_hbm.at[pl.ds(r0, _SUB)], x_buf)

                def row_body(r, _):
                    for g in range(d // _LANES):
                        off = g * _LANES
                        v = pe_buf[r, pl.ds(off, _LANES)]
                        plsc.addupdate(x_buf.at[r, pl.ds(off, _LANES)], v)
                    return 0

                lax.fori_loop(0, _SUB, row_body, 0, unroll=False)
                pltpu.sync_copy(x_buf, out_hbm.at[pl.ds(r0, _SUB)])
                return 0

            lax.fori_loop(0, b, batch_body, 0, unroll=False)
            return 0

        lax.fori_loop(0, nsub, sub_body, 0, unroll=False)

    return k


def kernel(x, pe_weight):
    b, l, d = x.shape
    xf = x.reshape(b * l, d)
    out = _make_sc_kernel(b, l, d, x.dtype)(xf, pe_weight)
    return out.reshape(b, l, d)
